# baseline (speedup 1.0000x reference)
import functools

import jax
import jax.numpy as jnp
from jax import lax
from jax.experimental import pallas as pl
from jax.experimental.pallas import tpu as pltpu
from jax.experimental.pallas import tpu_sc as plsc

N = 10000
D = 128
E = 320000
K = 10
ALPHA = 0.1

NTILES = 16
NCORES = 2
NW = NCORES * NTILES
NP = 10240
RPT = NP // NTILES
CH = 128
GC = 16
CPW = 80
NG = CPW // GC
EP = CPW * NW * CH
DUMP = NP - 1

_mesh = plsc.VectorSubcoreMesh(
    core_axis_name="c", subcore_axis_name="s", num_cores=NCORES)


@functools.partial(
    pl.kernel,
    out_type=jax.ShapeDtypeStruct((NCORES, NP, D), jnp.float32),
    mesh=_mesh,
    scratch_types=[
        pltpu.VMEM_SHARED((NP, D), jnp.float32),
        pltpu.VMEM((2, GC, CH), jnp.int32),
        pltpu.VMEM((2, GC, CH), jnp.int32),
        pltpu.VMEM((2, CH, D), jnp.float32),
        pltpu.SemaphoreType.DMA,
        pltpu.SemaphoreType.DMA,
        pltpu.SemaphoreType.DMA,
    ],
)
def _step_kernel(q_hbm, colp_hbm, rowp_hbm, z_hbm, pp_hbm,
                 pacc, cbufg, rbufg, gbuf, isem, gsem, ssem):
    c = lax.axis_index("c")
    s = lax.axis_index("s")
    w = c * NTILES + s
    r0 = s * RPT

    @pl.when(c == 0)
    def _():
        pltpu.sync_copy(q_hbm.at[pl.ds(r0, RPT)], pacc.at[pl.ds(r0, RPT)])

    @pl.when(c != 0)
    def _():
        pltpu.sync_copy(z_hbm.at[pl.ds(r0, RPT)], pacc.at[pl.ds(r0, RPT)])

    def _load_idx(grp):
        slot = grp % 2
        return (
            pltpu.async_copy(colp_hbm.at[w, pl.ds(grp * GC, GC)],
                             cbufg.at[slot], isem),
            pltpu.async_copy(rowp_hbm.at[w, pl.ds(grp * GC, GC)],
                             rbufg.at[slot], isem),
        )

    ivd = [None] * NG
    ivd[0] = _load_idx(0)
    for dsc in ivd[0]:
        dsc.wait()
    plsc.subcore_barrier()

    def _gather(j):
        grp, k = divmod(j, GC)
        return pltpu.async_copy(q_hbm.at[cbufg.at[grp % 2, k]],
                                gbuf.at[j % 2], gsem)

    gd = [None] * CPW
    sd = [None] * CPW
    gd[0] = _gather(0)
    for j in range(CPW):
        grp, k = divmod(j, GC)
        gd[j].wait()
        if k == 1 and grp + 1 < NG:
            ivd[grp + 1] = _load_idx(grp + 1)
        sd[j] = pltpu.async_copy(gbuf.at[j % 2], pacc.at[rbufg.at[grp % 2, k]],
                                 ssem, add=True)
        if j + 1 < CPW:
            g1, k1 = divmod(j + 1, GC)
            if k1 == 0:
                for dsc in ivd[g1]:
                    dsc.wait()
            if j >= 1:
                sd[j - 1].wait()
            gd[j + 1] = _gather(j + 1)
    sd[CPW - 2].wait()
    sd[CPW - 1].wait()
    plsc.subcore_barrier()
    pltpu.sync_copy(pacc.at[pl.ds(r0, RPT)], pp_hbm.at[c, pl.ds(r0, RPT)])


def _prep_body(x_ref, degw_ref, q0_ref, dinv2_ref, sdeg_ref):
    deg = degw_ref[0, :, 0:1] + degw_ref[1, :, 0:1]
    dinv = lax.rsqrt(deg)
    q0_ref[...] = x_ref[...] * dinv
    dinv2_ref[...] = 1.0 / deg
    sdeg_ref[...] = deg * dinv


_prep = pl.pallas_call(
    _prep_body,
    out_shape=(
        jax.ShapeDtypeStruct((NP, D), jnp.float32),
        jax.ShapeDtypeStruct((NP, 1), jnp.float32),
        jax.ShapeDtypeStruct((NP, 1), jnp.float32),
    ),
)


def _finalize_body(pp_ref, dinv2_ref, q_ref):
    q_ref[...] = (pp_ref[0] + pp_ref[1]) * dinv2_ref[...]


_finalize = pl.pallas_call(
    _finalize_body,
    out_shape=jax.ShapeDtypeStruct((NP, D), jnp.float32),
)

_BR = 1280


def _final_body(x_ref, sdeg_ref, w_ref, b_ref, *qs_out):
    qs, out_ref = qs_out[:-1], qs_out[-1]
    acc = qs[0][...]
    for qr in qs[1:]:
        acc = acc + qr[...]
    t = ((1.0 - ALPHA) / K) * sdeg_ref[...] * acc + ALPHA * x_ref[...]
    out_ref[...] = jnp.dot(t, w_ref[...],
                           preferred_element_type=jnp.float32) + b_ref[...]


_final = pl.pallas_call(
    _final_body,
    grid=(NP // _BR,),
    in_specs=[
        pl.BlockSpec((_BR, D), lambda i: (i, 0)),
        pl.BlockSpec((_BR, 1), lambda i: (i, 0)),
        pl.BlockSpec((D, D), lambda i: (0, 0)),
        pl.BlockSpec((1, D), lambda i: (0, 0)),
    ] + [pl.BlockSpec((_BR, D), lambda i: (i, 0)) for _ in range(K)],
    out_specs=pl.BlockSpec((_BR, D), lambda i: (i, 0)),
    out_shape=jax.ShapeDtypeStruct((NP, D), jnp.float32),
)


@functools.partial(
    pl.kernel,
    out_type=jax.ShapeDtypeStruct((NCORES, NP, D), jnp.float32),
    mesh=_mesh,
    scratch_types=[
        pltpu.VMEM_SHARED((NP, D), jnp.float32),
        pltpu.VMEM((2, GC, CH), jnp.int32),
        pltpu.VMEM((CH, D), jnp.float32),
        pltpu.SemaphoreType.DMA,
        pltpu.SemaphoreType.DMA,
    ],
)
def _deg_kernel(ones_hbm, rowp_hbm, z_hbm, pp_hbm,
                pacc, rbufg, onesb, isem, ssem):
    c = lax.axis_index("c")
    s = lax.axis_index("s")
    w = c * NTILES + s
    r0 = s * RPT

    @pl.when(c == 0)
    def _():
        pltpu.sync_copy(ones_hbm.at[pl.ds(r0, RPT)], pacc.at[pl.ds(r0, RPT)])

    @pl.when(c != 0)
    def _():
        pltpu.sync_copy(z_hbm.at[pl.ds(r0, RPT)], pacc.at[pl.ds(r0, RPT)])

    pltpu.sync_copy(ones_hbm.at[pl.ds(0, CH)], onesb)

    def _load_idx(grp):
        return pltpu.async_copy(rowp_hbm.at[w, pl.ds(grp * GC, GC)],
                                rbufg.at[grp % 2], isem)

    ivd = [None] * NG
    ivd[0] = _load_idx(0)
    ivd[0].wait()
    plsc.subcore_barrier()

    sd = [None] * CPW
    for j in range(CPW):
        grp, k = divmod(j, GC)
        if k == 4 and grp + 1 < NG:
            ivd[grp + 1] = _load_idx(grp + 1)
        if k == GC - 1 and grp + 1 < NG:
            ivd[grp + 1].wait()
        sd[j] = pltpu.async_copy(onesb, pacc.at[rbufg.at[grp % 2, k]],
                                 ssem, add=True)
        if j >= 4:
            sd[j - 4].wait()
    for j in range(CPW - 4, CPW):
        sd[j].wait()
    plsc.subcore_barrier()
    pltpu.sync_copy(pacc.at[pl.ds(r0, RPT)], pp_hbm.at[c, pl.ds(r0, RPT)])


H2 = NP // 2
TPC = 16
CPW2 = EP // TPC // CH
CAP2 = (CPW2 + 1) * CH
SLOTS2 = -(-(CPW2 + 1) // TPC)


@functools.partial(
    pl.kernel,
    out_type=(
        jax.ShapeDtypeStruct((NCORES, TPC, CAP2), jnp.int32),
        jax.ShapeDtypeStruct((NCORES, TPC, CAP2), jnp.int32),
        jax.ShapeDtypeStruct((NCORES, TPC, 16), jnp.int32),
    ),
    mesh=_mesh,
    scratch_types=[
        pltpu.VMEM((CPW2, CH), jnp.int32),
        pltpu.VMEM((CPW2, CH), jnp.int32),
        pltpu.VMEM((CAP2,), jnp.int32),
        pltpu.VMEM((CAP2,), jnp.int32),
        pltpu.VMEM((16,), jnp.int32),
        pltpu.SemaphoreType.DMA,
    ],
    compiler_params=pltpu.CompilerParams(needs_layout_passes=False),
)
def _part_kernel(colp16_hbm, rowp16_hbm, stc_hbm, str_hbm, cnt_hbm,
                 cbuf, rbuf, lc, lr, cntb, sem):
    c = lax.axis_index("c")
    s = lax.axis_index("s")
    pltpu.sync_copy(colp16_hbm.at[s], cbuf)
    pltpu.sync_copy(rowp16_hbm.at[s], rbuf)
    lo = c * H2
    lane = lax.iota(jnp.int32, 16)

    lov = jnp.full((16,), lo, jnp.int32)
    hiv = lov + H2

    def body(i, off):
        jj = i // 8
        kk = (i % 8) * 16
        cv = cbuf[jj, pl.ds(kk, 16)]
        rv = rbuf[jj, pl.ds(kk, 16)]
        mask = (rv >= lov) & (rv < hiv)
        mi = jnp.where(mask, jnp.full((16,), 1, jnp.int32),
                       jnp.full((16,), 0, jnp.int32))
        cum = plsc.cumsum(mi)
        pos = jnp.full((16,), off, jnp.int32) + cum - jnp.full(
            (16,), 1, jnp.int32)
        plsc.store_scatter(lc, [pos], cv, mask=mask)
        plsc.store_scatter(lr, [pos], rv, mask=mask)
        return off + jnp.sum(mi)

    off = lax.fori_loop(0, CPW2 * 8, body, jnp.int32(0))
    for t in range(8):
        pr = jnp.full((16,), lo + 5000, jnp.int32) + ((lane + 16 * t) % 120)
        pc = (lane + 16 * t) % 4096
        lr[pl.ds(off + 16 * t, 16)] = pr
        lc[pl.ds(off + 16 * t, 16)] = pc
    nch = (off + CH - 1) // CH
    cntb[...] = jnp.where(lane == 0, jnp.full((16,), nch, jnp.int32),
                          jnp.full((16,), 0, jnp.int32))
    pltpu.sync_copy(lc, stc_hbm.at[c, s])
    pltpu.sync_copy(lr, str_hbm.at[c, s])
    pltpu.sync_copy(cntb, cnt_hbm.at[c, s])


@functools.partial(
    pl.kernel,
    out_type=jax.ShapeDtypeStruct((NP, D), jnp.float32),
    mesh=_mesh,
    scratch_types=[
        pltpu.VMEM_SHARED((NP, D), jnp.float32),
        pltpu.VMEM((TPC, 16), jnp.int32),
        pltpu.VMEM((2, CH), jnp.int32),
        pltpu.VMEM((2, CH), jnp.int32),
        pltpu.VMEM((2, CH, D), jnp.float32),
        pltpu.SemaphoreType.DMA,
        pltpu.SemaphoreType.DMA,
        pltpu.SemaphoreType.DMA,
    ],
)
def _step2_kernel(q_hbm, stc_hbm, str_hbm, cnt_hbm, p_hbm,
                  pacc, countb, idxc, idxr, gbuf, isem, gsem, ssem):
    c = lax.axis_index("c")
    s = lax.axis_index("s")
    my0 = c * H2 + s * (H2 // NTILES)
    mysl = pl.ds(my0, H2 // NTILES)
    pltpu.sync_copy(q_hbm.at[mysl], pacc.at[mysl])
    pltpu.sync_copy(cnt_hbm.at[c], countb)
    plsc.subcore_barrier()

    def region(r, carry):
        nch = countb[r, pl.ds(0, 16)][0]
        for p_ in range(-(-SLOTS2 // 2)):
            jj0 = s + NTILES * (2 * p_)
            jj1 = s + NTILES * (2 * p_ + 1)
            a0 = jj0 < nch
            a1 = jj1 < nch
            dic0 = pltpu.make_async_copy(
                stc_hbm.at[c, r, pl.ds(jj0 * CH, CH)], idxc.at[0], isem)
            dir0 = pltpu.make_async_copy(
                str_hbm.at[c, r, pl.ds(jj0 * CH, CH)], idxr.at[0], isem)
            dic1 = pltpu.make_async_copy(
                stc_hbm.at[c, r, pl.ds(jj1 * CH, CH)], idxc.at[1], isem)
            dir1 = pltpu.make_async_copy(
                str_hbm.at[c, r, pl.ds(jj1 * CH, CH)], idxr.at[1], isem)
            dg0 = pltpu.make_async_copy(q_hbm.at[idxc.at[0]], gbuf.at[0], gsem)
            dg1 = pltpu.make_async_copy(q_hbm.at[idxc.at[1]], gbuf.at[1], gsem)
            ds0 = pltpu.make_async_copy(gbuf.at[0], pacc.at[idxr.at[0]], ssem)
            ds1 = pltpu.make_async_copy(gbuf.at[1], pacc.at[idxr.at[1]], ssem)

            @pl.when(a0)
            def _():
                dic0.start()
                dir0.start()

            @pl.when(a1)
            def _():
                dic1.start()
                dir1.start()

            @pl.when(a0)
            def _():
                dic0.wait()
                dir0.wait()
                dg0.start()

            @pl.when(a1)
            def _():
                dic1.wait()
                dir1.wait()
                dg1.start()

            @pl.when(a0)
            def _():
                dg0.wait()
                ds0.start(add=True)

            @pl.when(a1)
            def _():
                dg1.wait()
                ds1.start(add=True)

            @pl.when(a0)
            def _():
                ds0.wait()

            @pl.when(a1)
            def _():
                ds1.wait()
        return carry

    lax.fori_loop(0, TPC, region, 0)
    plsc.subcore_barrier()
    pltpu.sync_copy(pacc.at[mysl], p_hbm.at[mysl])


def _finalize2_body(p_ref, dinv2_ref, q_ref):
    q_ref[...] = p_ref[...] * dinv2_ref[...]


_finalize2 = pl.pallas_call(
    _finalize2_body,
    out_shape=jax.ShapeDtypeStruct((NP, D), jnp.float32),
)


def kernel(x, edge_index, W0, b0):
    x_perm = jnp.concatenate([
        x[:5000], jnp.zeros((H2 - 5000, D), jnp.float32),
        x[5000:], jnp.zeros((NP - H2 - 5000, D), jnp.float32)])
    col0 = edge_index[1]
    row0 = edge_index[0]
    col0 = col0 + jnp.where(col0 >= 5000, H2 - 5000, 0).astype(jnp.int32)
    row0 = row0 + jnp.where(row0 >= 5000, H2 - 5000, 0).astype(jnp.int32)
    pad = EP - E
    padi = jnp.arange(pad, dtype=jnp.int32)
    d2 = padi % 240
    padrow = jnp.where(d2 < 120, 5000 + d2, H2 + 5000 + (d2 - 120))
    colp = jnp.concatenate([col0, padi % 5000]).reshape(NW, CPW, CH)
    rowp = jnp.concatenate([row0, padrow]).reshape(NW, CPW, CH)
    z = jnp.zeros((NP, D), jnp.float32)
    ones = jnp.ones((NP, D), jnp.float32)

    stc, str_, cnt = _part_kernel(colp.reshape(TPC, CPW2, CH),
                                  rowp.reshape(TPC, CPW2, CH))
    degw = _deg_kernel(ones, rowp, z)
    q, dinv2, sdeg = _prep(x_perm, degw)

    qs = []
    for _ in range(K):
        p = _step2_kernel(q, stc, str_, cnt)
        q = _finalize2(p, dinv2)
        qs.append(q)

    out = _final(x_perm, sdeg, W0, b0.reshape(1, D), *qs)
    return jnp.concatenate([out[:5000], out[H2:H2 + 5000]])


# speedup vs baseline: 1.1885x; 1.1885x over previous
import functools

import jax
import jax.numpy as jnp
from jax import lax
from jax.experimental import pallas as pl
from jax.experimental.pallas import tpu as pltpu
from jax.experimental.pallas import tpu_sc as plsc

N = 10000
D = 128
E = 320000
K = 10
ALPHA = 0.1

NTILES = 16
NCORES = 2
NW = NCORES * NTILES
NP = 10240
RPT = NP // NTILES
CH = 128
GC = 16
CPW = 80
NG = CPW // GC
EP = CPW * NW * CH
DUMP = NP - 1

_mesh = plsc.VectorSubcoreMesh(
    core_axis_name="c", subcore_axis_name="s", num_cores=NCORES)


@functools.partial(
    pl.kernel,
    out_type=jax.ShapeDtypeStruct((NCORES, NP, D), jnp.float32),
    mesh=_mesh,
    scratch_types=[
        pltpu.VMEM_SHARED((NP, D), jnp.float32),
        pltpu.VMEM((2, GC, CH), jnp.int32),
        pltpu.VMEM((2, GC, CH), jnp.int32),
        pltpu.VMEM((2, CH, D), jnp.float32),
        pltpu.SemaphoreType.DMA,
        pltpu.SemaphoreType.DMA,
        pltpu.SemaphoreType.DMA,
    ],
)
def _step_kernel(q_hbm, colp_hbm, rowp_hbm, z_hbm, pp_hbm,
                 pacc, cbufg, rbufg, gbuf, isem, gsem, ssem):
    c = lax.axis_index("c")
    s = lax.axis_index("s")
    w = c * NTILES + s
    r0 = s * RPT

    @pl.when(c == 0)
    def _():
        pltpu.sync_copy(q_hbm.at[pl.ds(r0, RPT)], pacc.at[pl.ds(r0, RPT)])

    @pl.when(c != 0)
    def _():
        pltpu.sync_copy(z_hbm.at[pl.ds(r0, RPT)], pacc.at[pl.ds(r0, RPT)])

    def _load_idx(grp):
        slot = grp % 2
        return (
            pltpu.async_copy(colp_hbm.at[w, pl.ds(grp * GC, GC)],
                             cbufg.at[slot], isem),
            pltpu.async_copy(rowp_hbm.at[w, pl.ds(grp * GC, GC)],
                             rbufg.at[slot], isem),
        )

    ivd = [None] * NG
    ivd[0] = _load_idx(0)
    for dsc in ivd[0]:
        dsc.wait()
    plsc.subcore_barrier()

    def _gather(j):
        grp, k = divmod(j, GC)
        return pltpu.async_copy(q_hbm.at[cbufg.at[grp % 2, k]],
                                gbuf.at[j % 2], gsem)

    gd = [None] * CPW
    sd = [None] * CPW
    gd[0] = _gather(0)
    for j in range(CPW):
        grp, k = divmod(j, GC)
        gd[j].wait()
        if k == 1 and grp + 1 < NG:
            ivd[grp + 1] = _load_idx(grp + 1)
        sd[j] = pltpu.async_copy(gbuf.at[j % 2], pacc.at[rbufg.at[grp % 2, k]],
                                 ssem, add=True)
        if j + 1 < CPW:
            g1, k1 = divmod(j + 1, GC)
            if k1 == 0:
                for dsc in ivd[g1]:
                    dsc.wait()
            if j >= 1:
                sd[j - 1].wait()
            gd[j + 1] = _gather(j + 1)
    sd[CPW - 2].wait()
    sd[CPW - 1].wait()
    plsc.subcore_barrier()
    pltpu.sync_copy(pacc.at[pl.ds(r0, RPT)], pp_hbm.at[c, pl.ds(r0, RPT)])


def _prep_body(x_ref, degw_ref, q0_ref, dinv2_ref, sdeg_ref):
    deg = degw_ref[0, :, 0:1] + degw_ref[1, :, 0:1]
    dinv = lax.rsqrt(deg)
    q0_ref[...] = x_ref[...] * dinv
    dinv2_ref[...] = 1.0 / deg
    sdeg_ref[...] = deg * dinv


_prep = pl.pallas_call(
    _prep_body,
    out_shape=(
        jax.ShapeDtypeStruct((NP, D), jnp.float32),
        jax.ShapeDtypeStruct((NP, 1), jnp.float32),
        jax.ShapeDtypeStruct((NP, 1), jnp.float32),
    ),
)


def _finalize_body(pp_ref, dinv2_ref, q_ref):
    q_ref[...] = (pp_ref[0] + pp_ref[1]) * dinv2_ref[...]


_finalize = pl.pallas_call(
    _finalize_body,
    out_shape=jax.ShapeDtypeStruct((NP, D), jnp.float32),
)

_BR = 1280


def _final_body(x_ref, sdeg_ref, w_ref, b_ref, *qs_out):
    qs, out_ref = qs_out[:-1], qs_out[-1]
    acc = qs[0][...]
    for qr in qs[1:]:
        acc = acc + qr[...]
    t = ((1.0 - ALPHA) / K) * sdeg_ref[...] * acc + ALPHA * x_ref[...]
    out_ref[...] = jnp.dot(t, w_ref[...],
                           preferred_element_type=jnp.float32) + b_ref[...]


_final = pl.pallas_call(
    _final_body,
    grid=(NP // _BR,),
    in_specs=[
        pl.BlockSpec((_BR, D), lambda i: (i, 0)),
        pl.BlockSpec((_BR, 1), lambda i: (i, 0)),
        pl.BlockSpec((D, D), lambda i: (0, 0)),
        pl.BlockSpec((1, D), lambda i: (0, 0)),
    ] + [pl.BlockSpec((_BR, D), lambda i: (i, 0)) for _ in range(K)],
    out_specs=pl.BlockSpec((_BR, D), lambda i: (i, 0)),
    out_shape=jax.ShapeDtypeStruct((NP, D), jnp.float32),
)


@functools.partial(
    pl.kernel,
    out_type=jax.ShapeDtypeStruct((NCORES, NP, D), jnp.float32),
    mesh=_mesh,
    scratch_types=[
        pltpu.VMEM_SHARED((NP, D), jnp.float32),
        pltpu.VMEM((2, GC, CH), jnp.int32),
        pltpu.VMEM((CH, D), jnp.float32),
        pltpu.SemaphoreType.DMA,
        pltpu.SemaphoreType.DMA,
    ],
)
def _deg_kernel(ones_hbm, rowp_hbm, z_hbm, pp_hbm,
                pacc, rbufg, onesb, isem, ssem):
    c = lax.axis_index("c")
    s = lax.axis_index("s")
    w = c * NTILES + s
    r0 = s * RPT

    @pl.when(c == 0)
    def _():
        pltpu.sync_copy(ones_hbm.at[pl.ds(r0, RPT)], pacc.at[pl.ds(r0, RPT)])

    @pl.when(c != 0)
    def _():
        pltpu.sync_copy(z_hbm.at[pl.ds(r0, RPT)], pacc.at[pl.ds(r0, RPT)])

    pltpu.sync_copy(ones_hbm.at[pl.ds(0, CH)], onesb)

    def _load_idx(grp):
        return pltpu.async_copy(rowp_hbm.at[w, pl.ds(grp * GC, GC)],
                                rbufg.at[grp % 2], isem)

    ivd = [None] * NG
    ivd[0] = _load_idx(0)
    ivd[0].wait()
    plsc.subcore_barrier()

    sd = [None] * CPW
    for j in range(CPW):
        grp, k = divmod(j, GC)
        if k == 4 and grp + 1 < NG:
            ivd[grp + 1] = _load_idx(grp + 1)
        if k == GC - 1 and grp + 1 < NG:
            ivd[grp + 1].wait()
        sd[j] = pltpu.async_copy(onesb, pacc.at[rbufg.at[grp % 2, k]],
                                 ssem, add=True)
        if j >= 4:
            sd[j - 4].wait()
    for j in range(CPW - 4, CPW):
        sd[j].wait()
    plsc.subcore_barrier()
    pltpu.sync_copy(pacc.at[pl.ds(r0, RPT)], pp_hbm.at[c, pl.ds(r0, RPT)])


H2 = NP // 2
TPC = 16
CPW2 = EP // TPC // CH
CAPC = 192
SLOTS2 = 11


@functools.partial(
    pl.kernel,
    out_type=(
        jax.ShapeDtypeStruct((NCORES, TPC, CAPC, CH), jnp.int32),
        jax.ShapeDtypeStruct((NCORES, TPC, CAPC, CH), jnp.int32),
        jax.ShapeDtypeStruct((NCORES, TPC, 16), jnp.int32),
    ),
    mesh=_mesh,
    scratch_types=[
        pltpu.VMEM((CPW2, CH), jnp.int32),
        pltpu.VMEM((CPW2, CH), jnp.int32),
        pltpu.VMEM((CAPC, CH), jnp.int32),
        pltpu.VMEM((CAPC, CH), jnp.int32),
        pltpu.VMEM((16,), jnp.int32),
        pltpu.SemaphoreType.DMA,
    ],
    compiler_params=pltpu.CompilerParams(needs_layout_passes=False),
)
def _part_kernel(colp16_hbm, rowp16_hbm, stc_hbm, str_hbm, cnt_hbm,
                 cbuf, rbuf, lc, lr, cntb, sem):
    c = lax.axis_index("c")
    s = lax.axis_index("s")
    pltpu.sync_copy(colp16_hbm.at[s], cbuf)
    pltpu.sync_copy(rowp16_hbm.at[s], rbuf)
    lo = c * H2
    lane = lax.iota(jnp.int32, 16)
    lov = jnp.full((16,), lo, jnp.int32)
    hiv = lov + H2
    c127 = jnp.full((16,), 127, jnp.int32)

    def body(i, off):
        jj = i // 8
        kk = (i % 8) * 16
        cv = cbuf[jj, pl.ds(kk, 16)]
        rv = rbuf[jj, pl.ds(kk, 16)]
        mask = (rv >= lov) & (rv < hiv)
        mi = jnp.where(mask, jnp.full((16,), 1, jnp.int32),
                       jnp.full((16,), 0, jnp.int32))
        cum = plsc.cumsum(mi)
        pos = jnp.full((16,), off, jnp.int32) + cum - jnp.full(
            (16,), 1, jnp.int32)
        prow = lax.shift_right_logical(pos, 7)
        pcol = pos & c127
        plsc.store_scatter(lc, [prow, pcol], cv, mask=mask)
        plsc.store_scatter(lr, [prow, pcol], rv - lov, mask=mask)
        return off + jnp.sum(mi)

    off = lax.fori_loop(0, CPW2 * 8, body, jnp.int32(0))
    for t in range(8):
        pos = jnp.full((16,), off + 16 * t, jnp.int32) + lane
        prow = lax.shift_right_logical(pos, 7)
        pcol = pos & c127
        plsc.store_scatter(lr, [prow, pcol],
                           jnp.full((16,), 5000, jnp.int32)
                           + ((lane + 16 * t) % 120))
        plsc.store_scatter(lc, [prow, pcol], (lane + 16 * t) % 4096)
    nch = (off + CH - 1) // CH
    cntb[...] = jnp.where(lane == 0, jnp.full((16,), nch, jnp.int32),
                          jnp.full((16,), 0, jnp.int32))
    pltpu.sync_copy(lc, stc_hbm.at[c, s])
    pltpu.sync_copy(lr, str_hbm.at[c, s])
    pltpu.sync_copy(cntb, cnt_hbm.at[c, s])


@functools.partial(
    pl.kernel,
    out_type=jax.ShapeDtypeStruct((NP, D), jnp.float32),
    mesh=_mesh,
    scratch_types=[
        pltpu.VMEM_SHARED((H2, D), jnp.float32),
        pltpu.VMEM((TPC, 16), jnp.int32),
        pltpu.VMEM((24, CH), jnp.int32),
        pltpu.VMEM((24, CH), jnp.int32),
        pltpu.VMEM((4, CH, D), jnp.float32),
        pltpu.SemaphoreType.DMA,
        pltpu.SemaphoreType.DMA,
        pltpu.SemaphoreType.DMA,
    ],
)
def _step2_kernel(q_hbm, stc_hbm, str_hbm, cnt_hbm, p_hbm,
                  pacc, countb, idxc, idxr, gbuf, isem, gsem, ssem):
    c = lax.axis_index("c")
    s = lax.axis_index("s")
    rpt2 = H2 // NTILES
    pltpu.sync_copy(q_hbm.at[pl.ds(c * H2 + s * rpt2, rpt2)],
                    pacc.at[pl.ds(s * rpt2, rpt2)])
    pltpu.sync_copy(cnt_hbm.at[c], countb)
    plsc.subcore_barrier()

    def region(r, carry):
        nch = countb[r, pl.ds(0, 16)][0]
        spn = (nch + NTILES - 1) // NTILES
        start = s * spn
        cnt = jnp.maximum(0, jnp.minimum(spn, nch - start))
        start8 = pl.multiple_of((start // 8) * 8, 8)
        off8 = start - start8
        dic = pltpu.make_async_copy(stc_hbm.at[c, r, pl.ds(start8, 24)],
                                    idxc, isem)
        dir_ = pltpu.make_async_copy(str_hbm.at[c, r, pl.ds(start8, 24)],
                                     idxr, isem)

        @pl.when(cnt > 0)
        def _():
            dic.start()
            dir_.start()
            dic.wait()
            dir_.wait()

        gd = [pltpu.make_async_copy(q_hbm.at[idxc.at[off8 + u]], gbuf.at[u % 4],
                                    gsem) for u in range(SLOTS2)]
        sd = [pltpu.make_async_copy(gbuf.at[u % 4], pacc.at[idxr.at[off8 + u]],
                                    ssem) for u in range(SLOTS2)]
        for u in range(4):
            @pl.when(u < cnt)
            def _(u=u):
                gd[u].start()
        for u in range(SLOTS2):
            @pl.when(u < cnt)
            def _(u=u):
                gd[u].wait()
                sd[u].start(add=True)
                sd[u].wait()
            if u + 4 < SLOTS2:
                @pl.when(u + 4 < cnt)
                def _(u=u):
                    gd[u + 4].start()
        return carry

    lax.fori_loop(0, TPC, region, 0)
    plsc.subcore_barrier()
    pltpu.sync_copy(pacc.at[pl.ds(s * rpt2, rpt2)],
                    p_hbm.at[pl.ds(c * H2 + s * rpt2, rpt2)])


def _finalize2_body(p_ref, dinv2_ref, q_ref):
    q_ref[...] = p_ref[...] * dinv2_ref[...]


_finalize2 = pl.pallas_call(
    _finalize2_body,
    out_shape=jax.ShapeDtypeStruct((NP, D), jnp.float32),
)


def kernel(x, edge_index, W0, b0):
    x_perm = jnp.concatenate([
        x[:5000], jnp.zeros((H2 - 5000, D), jnp.float32),
        x[5000:], jnp.zeros((NP - H2 - 5000, D), jnp.float32)])
    col0 = edge_index[1]
    row0 = edge_index[0]
    col0 = col0 + jnp.where(col0 >= 5000, H2 - 5000, 0).astype(jnp.int32)
    row0 = row0 + jnp.where(row0 >= 5000, H2 - 5000, 0).astype(jnp.int32)
    pad = EP - E
    padi = jnp.arange(pad, dtype=jnp.int32)
    d2 = padi % 240
    padrow = jnp.where(d2 < 120, 5000 + d2, H2 + 5000 + (d2 - 120))
    colp = jnp.concatenate([col0, padi % 5000]).reshape(NW, CPW, CH)
    rowp = jnp.concatenate([row0, padrow]).reshape(NW, CPW, CH)
    z = jnp.zeros((NP, D), jnp.float32)
    ones = jnp.ones((NP, D), jnp.float32)

    stc, str_, cnt = _part_kernel(colp.reshape(TPC, CPW2, CH),
                                  rowp.reshape(TPC, CPW2, CH))
    degw = _deg_kernel(ones, rowp, z)
    q, dinv2, sdeg = _prep(x_perm, degw)

    qs = []
    for _ in range(K):
        p = _step2_kernel(q, stc, str_, cnt)
        q = _finalize2(p, dinv2)
        qs.append(q)

    out = _final(x_perm, sdeg, W0, b0.reshape(1, D), *qs)
    return jnp.concatenate([out[:5000], out[H2:H2 + 5000]])


# speedup vs baseline: 1.2372x; 1.0409x over previous
import functools

import jax
import jax.numpy as jnp
from jax import lax
from jax.experimental import pallas as pl
from jax.experimental.pallas import tpu as pltpu
from jax.experimental.pallas import tpu_sc as plsc

N = 10000
D = 128
E = 320000
K = 10
ALPHA = 0.1

NTILES = 16
NCORES = 2
NW = NCORES * NTILES
NP = 10240
RPT = NP // NTILES
CH = 128
GC = 16
CPW = 80
NG = CPW // GC
EP = CPW * NW * CH
DUMP = NP - 1

_mesh = plsc.VectorSubcoreMesh(
    core_axis_name="c", subcore_axis_name="s", num_cores=NCORES)


@functools.partial(
    pl.kernel,
    out_type=jax.ShapeDtypeStruct((NCORES, NP, D), jnp.float32),
    mesh=_mesh,
    scratch_types=[
        pltpu.VMEM_SHARED((NP, D), jnp.float32),
        pltpu.VMEM((2, GC, CH), jnp.int32),
        pltpu.VMEM((2, GC, CH), jnp.int32),
        pltpu.VMEM((2, CH, D), jnp.float32),
        pltpu.SemaphoreType.DMA,
        pltpu.SemaphoreType.DMA,
        pltpu.SemaphoreType.DMA,
    ],
)
def _step_kernel(q_hbm, colp_hbm, rowp_hbm, z_hbm, pp_hbm,
                 pacc, cbufg, rbufg, gbuf, isem, gsem, ssem):
    c = lax.axis_index("c")
    s = lax.axis_index("s")
    w = c * NTILES + s
    r0 = s * RPT

    @pl.when(c == 0)
    def _():
        pltpu.sync_copy(q_hbm.at[pl.ds(r0, RPT)], pacc.at[pl.ds(r0, RPT)])

    @pl.when(c != 0)
    def _():
        pltpu.sync_copy(z_hbm.at[pl.ds(r0, RPT)], pacc.at[pl.ds(r0, RPT)])

    def _load_idx(grp):
        slot = grp % 2
        return (
            pltpu.async_copy(colp_hbm.at[w, pl.ds(grp * GC, GC)],
                             cbufg.at[slot], isem),
            pltpu.async_copy(rowp_hbm.at[w, pl.ds(grp * GC, GC)],
                             rbufg.at[slot], isem),
        )

    ivd = [None] * NG
    ivd[0] = _load_idx(0)
    for dsc in ivd[0]:
        dsc.wait()
    plsc.subcore_barrier()

    def _gather(j):
        grp, k = divmod(j, GC)
        return pltpu.async_copy(q_hbm.at[cbufg.at[grp % 2, k]],
                                gbuf.at[j % 2], gsem)

    gd = [None] * CPW
    sd = [None] * CPW
    gd[0] = _gather(0)
    for j in range(CPW):
        grp, k = divmod(j, GC)
        gd[j].wait()
        if k == 1 and grp + 1 < NG:
            ivd[grp + 1] = _load_idx(grp + 1)
        sd[j] = pltpu.async_copy(gbuf.at[j % 2], pacc.at[rbufg.at[grp % 2, k]],
                                 ssem, add=True)
        if j + 1 < CPW:
            g1, k1 = divmod(j + 1, GC)
            if k1 == 0:
                for dsc in ivd[g1]:
                    dsc.wait()
            if j >= 1:
                sd[j - 1].wait()
            gd[j + 1] = _gather(j + 1)
    sd[CPW - 2].wait()
    sd[CPW - 1].wait()
    plsc.subcore_barrier()
    pltpu.sync_copy(pacc.at[pl.ds(r0, RPT)], pp_hbm.at[c, pl.ds(r0, RPT)])


def _prep_body(x_ref, degw_ref, q0_ref, dinv2_ref, sdeg_ref):
    deg = degw_ref[0, :, 0:1] + degw_ref[1, :, 0:1]
    dinv = lax.rsqrt(deg)
    q0_ref[...] = x_ref[...] * dinv
    dinv2_ref[...] = 1.0 / deg
    sdeg_ref[...] = deg * dinv


_prep = pl.pallas_call(
    _prep_body,
    out_shape=(
        jax.ShapeDtypeStruct((NP, D), jnp.float32),
        jax.ShapeDtypeStruct((NP, 1), jnp.float32),
        jax.ShapeDtypeStruct((NP, 1), jnp.float32),
    ),
)


def _finalize_body(pp_ref, dinv2_ref, q_ref):
    q_ref[...] = (pp_ref[0] + pp_ref[1]) * dinv2_ref[...]


_finalize = pl.pallas_call(
    _finalize_body,
    out_shape=jax.ShapeDtypeStruct((NP, D), jnp.float32),
)

_BR = 1280


def _final_body(x_ref, sdeg_ref, w_ref, b_ref, *qs_out):
    qs, out_ref = qs_out[:-1], qs_out[-1]
    acc = qs[0][...]
    for qr in qs[1:]:
        acc = acc + qr[...]
    t = ((1.0 - ALPHA) / K) * sdeg_ref[...] * acc + ALPHA * x_ref[...]
    out_ref[...] = jnp.dot(t, w_ref[...],
                           preferred_element_type=jnp.float32) + b_ref[...]


_final = pl.pallas_call(
    _final_body,
    grid=(NP // _BR,),
    in_specs=[
        pl.BlockSpec((_BR, D), lambda i: (i, 0)),
        pl.BlockSpec((_BR, 1), lambda i: (i, 0)),
        pl.BlockSpec((D, D), lambda i: (0, 0)),
        pl.BlockSpec((1, D), lambda i: (0, 0)),
    ] + [pl.BlockSpec((_BR, D), lambda i: (i, 0)) for _ in range(K)],
    out_specs=pl.BlockSpec((_BR, D), lambda i: (i, 0)),
    out_shape=jax.ShapeDtypeStruct((NP, D), jnp.float32),
)


@functools.partial(
    pl.kernel,
    out_type=jax.ShapeDtypeStruct((NCORES, NP, D), jnp.float32),
    mesh=_mesh,
    scratch_types=[
        pltpu.VMEM_SHARED((NP, D), jnp.float32),
        pltpu.VMEM((2, GC, CH), jnp.int32),
        pltpu.VMEM((CH, D), jnp.float32),
        pltpu.SemaphoreType.DMA,
        pltpu.SemaphoreType.DMA,
    ],
)
def _deg_kernel(ones_hbm, rowp_hbm, z_hbm, pp_hbm,
                pacc, rbufg, onesb, isem, ssem):
    c = lax.axis_index("c")
    s = lax.axis_index("s")
    w = c * NTILES + s
    r0 = s * RPT

    @pl.when(c == 0)
    def _():
        pltpu.sync_copy(ones_hbm.at[pl.ds(r0, RPT)], pacc.at[pl.ds(r0, RPT)])

    @pl.when(c != 0)
    def _():
        pltpu.sync_copy(z_hbm.at[pl.ds(r0, RPT)], pacc.at[pl.ds(r0, RPT)])

    pltpu.sync_copy(ones_hbm.at[pl.ds(0, CH)], onesb)

    def _load_idx(grp):
        return pltpu.async_copy(rowp_hbm.at[w, pl.ds(grp * GC, GC)],
                                rbufg.at[grp % 2], isem)

    ivd = [None] * NG
    ivd[0] = _load_idx(0)
    ivd[0].wait()
    plsc.subcore_barrier()

    sd = [None] * CPW
    for j in range(CPW):
        grp, k = divmod(j, GC)
        if k == 4 and grp + 1 < NG:
            ivd[grp + 1] = _load_idx(grp + 1)
        if k == GC - 1 and grp + 1 < NG:
            ivd[grp + 1].wait()
        sd[j] = pltpu.async_copy(onesb, pacc.at[rbufg.at[grp % 2, k]],
                                 ssem, add=True)
        if j >= 4:
            sd[j - 4].wait()
    for j in range(CPW - 4, CPW):
        sd[j].wait()
    plsc.subcore_barrier()
    pltpu.sync_copy(pacc.at[pl.ds(r0, RPT)], pp_hbm.at[c, pl.ds(r0, RPT)])


H2 = NP // 2
TPC = 16
CPW2 = EP // TPC // CH
CAPC = 192
SLOTS2 = 11


@functools.partial(
    pl.kernel,
    out_type=(
        jax.ShapeDtypeStruct((NCORES, TPC, CAPC, CH), jnp.int32),
        jax.ShapeDtypeStruct((NCORES, TPC, CAPC, CH), jnp.int32),
        jax.ShapeDtypeStruct((NCORES, TPC, 16), jnp.int32),
    ),
    mesh=_mesh,
    scratch_types=[
        pltpu.VMEM((CPW2, CH), jnp.int32),
        pltpu.VMEM((CPW2, CH), jnp.int32),
        pltpu.VMEM((CAPC, CH), jnp.int32),
        pltpu.VMEM((CAPC, CH), jnp.int32),
        pltpu.VMEM((16,), jnp.int32),
        pltpu.SemaphoreType.DMA,
    ],
    compiler_params=pltpu.CompilerParams(needs_layout_passes=False),
)
def _part_kernel(colp16_hbm, rowp16_hbm, stc_hbm, str_hbm, cnt_hbm,
                 cbuf, rbuf, lc, lr, cntb, sem):
    c = lax.axis_index("c")
    s = lax.axis_index("s")
    pltpu.sync_copy(colp16_hbm.at[s], cbuf)
    pltpu.sync_copy(rowp16_hbm.at[s], rbuf)
    lo = c * H2
    lane = lax.iota(jnp.int32, 16)
    lov = jnp.full((16,), lo, jnp.int32)
    hiv = lov + H2
    c127 = jnp.full((16,), 127, jnp.int32)

    def body(i, off):
        jj = i // 8
        kk = (i % 8) * 16
        cv = cbuf[jj, pl.ds(kk, 16)]
        rv = rbuf[jj, pl.ds(kk, 16)]
        mask = (rv >= lov) & (rv < hiv)
        mi = jnp.where(mask, jnp.full((16,), 1, jnp.int32),
                       jnp.full((16,), 0, jnp.int32))
        cum = plsc.cumsum(mi)
        pos = jnp.full((16,), off, jnp.int32) + cum - jnp.full(
            (16,), 1, jnp.int32)
        prow = lax.shift_right_logical(pos, 7)
        pcol = pos & c127
        plsc.store_scatter(lc, [prow, pcol], cv, mask=mask)
        plsc.store_scatter(lr, [prow, pcol], rv - lov, mask=mask)
        return off + jnp.sum(mi)

    off = lax.fori_loop(0, CPW2 * 8, body, jnp.int32(0))
    for t in range(8):
        pos = jnp.full((16,), off + 16 * t, jnp.int32) + lane
        prow = lax.shift_right_logical(pos, 7)
        pcol = pos & c127
        plsc.store_scatter(lr, [prow, pcol],
                           jnp.full((16,), 5000, jnp.int32)
                           + ((lane + 16 * t) % 120))
        plsc.store_scatter(lc, [prow, pcol], (lane + 16 * t) % 4096)
    nch = (off + CH - 1) // CH
    cntb[...] = jnp.where(lane == 0, jnp.full((16,), nch, jnp.int32),
                          jnp.full((16,), 0, jnp.int32))
    pltpu.sync_copy(lc, stc_hbm.at[c, s])
    pltpu.sync_copy(lr, str_hbm.at[c, s])
    pltpu.sync_copy(cntb, cnt_hbm.at[c, s])


@functools.partial(
    pl.kernel,
    out_type=jax.ShapeDtypeStruct((NP, D), jnp.float32),
    mesh=_mesh,
    scratch_types=[
        pltpu.VMEM_SHARED((H2, D), jnp.float32),
        pltpu.VMEM((TPC, 16), jnp.int32),
        pltpu.VMEM((24, CH), jnp.int32),
        pltpu.VMEM((24, CH), jnp.int32),
        pltpu.VMEM((4, CH, D), jnp.float32),
        pltpu.SemaphoreType.DMA,
        pltpu.SemaphoreType.DMA,
        pltpu.SemaphoreType.DMA,
    ],
)
def _step2_kernel(q_hbm, stc_hbm, str_hbm, cnt_hbm, p_hbm,
                  pacc, countb, idxc, idxr, gbuf, isem, gsem, ssem):
    c = lax.axis_index("c")
    s = lax.axis_index("s")
    rpt2 = H2 // NTILES
    pltpu.sync_copy(q_hbm.at[pl.ds(c * H2 + s * rpt2, rpt2)],
                    pacc.at[pl.ds(s * rpt2, rpt2)])
    pltpu.sync_copy(cnt_hbm.at[c], countb)
    plsc.subcore_barrier()

    def region(r, carry):
        nch = countb[r, pl.ds(0, 16)][0]
        spn = (nch + NTILES - 1) // NTILES
        start = s * spn
        cnt = jnp.maximum(0, jnp.minimum(spn, nch - start))
        start8 = pl.multiple_of((start // 8) * 8, 8)
        off8 = start - start8
        dic = pltpu.make_async_copy(stc_hbm.at[c, r, pl.ds(start8, 24)],
                                    idxc, isem)
        dir_ = pltpu.make_async_copy(str_hbm.at[c, r, pl.ds(start8, 24)],
                                     idxr, isem)

        @pl.when(cnt > 0)
        def _():
            dic.start()
            dir_.start()
            dic.wait()
            dir_.wait()

        gd = [pltpu.make_async_copy(q_hbm.at[idxc.at[off8 + u]], gbuf.at[u % 4],
                                    gsem) for u in range(SLOTS2)]
        sd = [pltpu.make_async_copy(gbuf.at[u % 4], pacc.at[idxr.at[off8 + u]],
                                    ssem) for u in range(SLOTS2)]
        for u in range(3):
            @pl.when(u < cnt)
            def _(u=u):
                gd[u].start()
        for u in range(SLOTS2):
            @pl.when(u < cnt)
            def _(u=u):
                gd[u].wait()
                sd[u].start(add=True)
            if u >= 1:
                @pl.when(u - 1 < cnt)
                def _(u=u):
                    sd[u - 1].wait()
            if u + 3 < SLOTS2:
                @pl.when(u + 3 < cnt)
                def _(u=u):
                    gd[u + 3].start()

        @pl.when(SLOTS2 - 1 < cnt)
        def _():
            sd[SLOTS2 - 1].wait()
        return carry

    lax.fori_loop(0, TPC, region, 0)
    plsc.subcore_barrier()
    pltpu.sync_copy(pacc.at[pl.ds(s * rpt2, rpt2)],
                    p_hbm.at[pl.ds(c * H2 + s * rpt2, rpt2)])


def _finalize2_body(p_ref, dinv2_ref, q_ref):
    q_ref[...] = p_ref[...] * dinv2_ref[...]


_finalize2 = pl.pallas_call(
    _finalize2_body,
    out_shape=jax.ShapeDtypeStruct((NP, D), jnp.float32),
)


def kernel(x, edge_index, W0, b0):
    x_perm = jnp.concatenate([
        x[:5000], jnp.zeros((H2 - 5000, D), jnp.float32),
        x[5000:], jnp.zeros((NP - H2 - 5000, D), jnp.float32)])
    col0 = edge_index[1]
    row0 = edge_index[0]
    col0 = col0 + jnp.where(col0 >= 5000, H2 - 5000, 0).astype(jnp.int32)
    row0 = row0 + jnp.where(row0 >= 5000, H2 - 5000, 0).astype(jnp.int32)
    pad = EP - E
    padi = jnp.arange(pad, dtype=jnp.int32)
    d2 = padi % 240
    padrow = jnp.where(d2 < 120, 5000 + d2, H2 + 5000 + (d2 - 120))
    colp = jnp.concatenate([col0, padi % 5000]).reshape(NW, CPW, CH)
    rowp = jnp.concatenate([row0, padrow]).reshape(NW, CPW, CH)
    z = jnp.zeros((NP, D), jnp.float32)
    ones = jnp.ones((NP, D), jnp.float32)

    stc, str_, cnt = _part_kernel(colp.reshape(TPC, CPW2, CH),
                                  rowp.reshape(TPC, CPW2, CH))
    degw = _deg_kernel(ones, rowp, z)
    q, dinv2, sdeg = _prep(x_perm, degw)

    qs = []
    for _ in range(K):
        p = _step2_kernel(q, stc, str_, cnt)
        q = _finalize2(p, dinv2)
        qs.append(q)

    out = _final(x_perm, sdeg, W0, b0.reshape(1, D), *qs)
    return jnp.concatenate([out[:5000], out[H2:H2 + 5000]])
